# baseline (speedup 1.0000x reference)
import functools

import jax
import jax.numpy as jnp
from jax import lax
from jax.experimental import pallas as pl
from jax.experimental.pallas import tpu as pltpu
from jax.experimental.pallas import tpu_sc as plsc

B = 16384
D = 64
NC, NS = 2, 16
NW = NC * NS
BPW = B // NW
CHUNK = 256
NCHUNK = BPW // CHUNK
G = 16

N_USER = 1000000
SC_ROWS = 799872
HALF = SC_ROWS // 2


def _sc_gather(u3, ub, emb_a, emb_c, lin_u1, lin_a1, lin_c1,
               idx_u, idx_a, idx_c):
    mesh = plsc.VectorSubcoreMesh(core_axis_name="c", subcore_axis_name="s")
    f32 = jnp.float32
    out_types = (
        jax.ShapeDtypeStruct((B, D), f32),
        jax.ShapeDtypeStruct((B, D), f32),
        jax.ShapeDtypeStruct((B, D), f32),
        jax.ShapeDtypeStruct((B,), f32),
        jax.ShapeDtypeStruct((B,), f32),
        jax.ShapeDtypeStruct((B,), f32),
    )
    scratch = (
        [pltpu.VMEM((CHUNK,), jnp.int32) for _ in range(3)]
        + [pltpu.VMEM((CHUNK, D), f32) for _ in range(3)]
        + [pltpu.VMEM((CHUNK,), f32) for _ in range(3)]
        + [pltpu.SemaphoreType.DMA, pltpu.SemaphoreType.DMA]
    )

    @functools.partial(pl.kernel, mesh=mesh, out_type=out_types,
                       scratch_types=scratch)
    def k(u3_h, ub_h, ea_h, ec_h, lu_h, la_h, lc_h,
          iu_h, ia_h, ic_h,
          oeu, oea, oec, olu, ola, olc,
          viu, via, vic, veu, vea, vec, vlu, vla, vlc, sem, lsem):
        wid = lax.axis_index("s") * NC + lax.axis_index("c")
        base = wid * BPW
        for c in range(NCHUNK):
            off = base + c * CHUNK
            pltpu.sync_copy(iu_h.at[pl.ds(off, CHUNK)], viu)
            pltpu.sync_copy(ia_h.at[pl.ds(off, CHUNK)], via)
            pltpu.sync_copy(ic_h.at[pl.ds(off, CHUNK)], vic)
            hl = [pltpu.async_copy(lu_h.at[viu], vlu, lsem),
                  pltpu.async_copy(la_h.at[via], vla, lsem),
                  pltpu.async_copy(lc_h.at[vic], vlc, lsem)]

            @pl.loop(0, CHUNK // G)
            def _(g):
                vu = viu[pl.ds(g * G, G)]
                va = via[pl.ds(g * G, G)]
                vc = vic[pl.ds(g * G, G)]
                for j in range(G):
                    p = g * G + j
                    iu = vu[j]
                    in_sc = iu < SC_ROWS
                    hi = (iu >= HALF).astype(jnp.int32)
                    r2 = iu - hi * HALF

                    @pl.when(in_sc)
                    def _():
                        pltpu.async_copy(u3_h.at[hi].at[pl.ds(r2, 1)],
                                         veu.at[pl.ds(p, 1)], sem)

                    @pl.when(jnp.logical_not(in_sc))
                    def _():
                        pltpu.async_copy(ub_h.at[pl.ds(iu - SC_ROWS, 1)],
                                         veu.at[pl.ds(p, 1)], sem)

                    pltpu.async_copy(
                        ea_h.at[pl.ds(va[j], 1)], vea.at[pl.ds(p, 1)], sem)
                    pltpu.async_copy(
                        ec_h.at[pl.ds(vc[j], 1)], vec.at[pl.ds(p, 1)], sem)

            pltpu.make_async_copy(ea_h.at[pl.ds(0, CHUNK)], veu, sem).wait()
            pltpu.make_async_copy(ea_h.at[pl.ds(0, CHUNK)], vea, sem).wait()
            pltpu.make_async_copy(ea_h.at[pl.ds(0, CHUNK)], vec, sem).wait()
            for h in hl:
                h.wait()
            pltpu.sync_copy(veu, oeu.at[pl.ds(off, CHUNK)])
            pltpu.sync_copy(vea, oea.at[pl.ds(off, CHUNK)])
            pltpu.sync_copy(vec, oec.at[pl.ds(off, CHUNK)])
            pltpu.sync_copy(vlu, olu.at[pl.ds(off, CHUNK)])
            pltpu.sync_copy(vla, ola.at[pl.ds(off, CHUNK)])
            pltpu.sync_copy(vlc, olc.at[pl.ds(off, CHUNK)])

    return k(u3, ub, emb_a, emb_c, lin_u1, lin_a1, lin_c1,
             idx_u, idx_a, idx_c)


def _tc_body(eu_r, ea_r, ec_r, lt_r,
             W1_r, b1_r, W2_r, b2_r, w3_r, off_r, o_r):
    eu = eu_r[...]
    ea = ea_r[...]
    ec = ec_r[...]
    s = eu + ea + ec
    sum_sq = jnp.sum(s * s, axis=1, keepdims=True)
    sq_sum = jnp.sum(eu * eu + ea * ea + ec * ec, axis=1, keepdims=True)
    interaction = 0.5 * (sum_sq - sq_sum)
    W1 = W1_r[...]
    f32 = jnp.float32
    h = (jnp.dot(eu, W1[0:D, :], preferred_element_type=f32)
         + jnp.dot(ea, W1[D:2 * D, :], preferred_element_type=f32)
         + jnp.dot(ec, W1[2 * D:3 * D, :], preferred_element_type=f32)
         + b1_r[...])
    h = jnp.maximum(h, 0.0)
    h = jnp.dot(h, W2_r[...], preferred_element_type=f32) + b2_r[...]
    h = jnp.maximum(h, 0.0)
    deep = jnp.sum(h * w3_r[...], axis=1, keepdims=True)
    o_r[...] = jax.nn.sigmoid(off_r[...] + lt_r[...] + interaction + deep)


def _tc_dense(eu, ea, ec, lt, W1, b1, W2, b2, w3, off):
    grid = (B // BLK,)
    bs_emb = pl.BlockSpec((BLK, D), lambda i: (i, 0))
    bs_col = pl.BlockSpec((BLK, 1), lambda i: (i, 0))
    rep = lambda shape: pl.BlockSpec(shape, lambda i: (0, 0))
    return pl.pallas_call(
        _tc_body,
        grid=grid,
        in_specs=[
            bs_emb, bs_emb, bs_emb, bs_col,
            rep((3 * D, 128)), rep((1, 128)),
            rep((128, D)), rep((1, D)),
            rep((1, D)), rep((1, 1)),
        ],
        out_specs=pl.BlockSpec((BLK, 1), lambda i: (i, 0)),
        out_shape=jax.ShapeDtypeStruct((B, 1), jnp.float32),
    )(eu, ea, ec, lt, W1, b1, W2, b2, w3, off)


BLK = 1024


def kernel(user_id, ad_id, context_id, lin_user, lin_ad, lin_ctx,
           emb_user, emb_ad, emb_ctx, bias, W1, b1, W2, b2, W3, b3):
    lu1 = lin_user.reshape(-1)
    la1 = lin_ad.reshape(-1)
    lc1 = lin_ctx.reshape(-1)
    uid = user_id.astype(jnp.int32)
    aid = ad_id.astype(jnp.int32)
    cid = context_id.astype(jnp.int32)

    u3 = emb_user[:SC_ROWS].reshape(2, HALF, D)
    ub = emb_user[SC_ROWS:]

    geu, gea, gec, glu, gla, glc = _sc_gather(
        u3, ub, emb_ad, emb_ctx, lu1, la1, lc1, uid, aid, cid)

    lt = (glu + gla + glc).reshape(B, 1)
    off = (bias + b3).reshape(1, 1)
    out = _tc_dense(geu, gea, gec, lt,
                    W1, b1.reshape(1, 128), W2, b2.reshape(1, D),
                    W3.reshape(1, D), off)
    return out.reshape(B)


# speedup vs baseline: 1.6049x; 1.6049x over previous
import functools

import jax
import jax.numpy as jnp
from jax import lax
from jax.experimental import pallas as pl
from jax.experimental.pallas import tpu as pltpu
from jax.experimental.pallas import tpu_sc as plsc

B = 16384
D = 64
NC, NS = 2, 16
NW = NC * NS
BPW = B // NW
CHUNK = 256
NCHUNK = BPW // CHUNK
G = 16

N_USER = 1000000
HALF = N_USER // 2


def _sc_gather(u3, emb_a, emb_c, lin_u1, lin_a1, lin_c1,
               idx_u, idx_a, idx_c):
    mesh = plsc.VectorSubcoreMesh(core_axis_name="c", subcore_axis_name="s")
    f32 = jnp.float32
    out_types = (
        jax.ShapeDtypeStruct((B, D), f32),
        jax.ShapeDtypeStruct((B, D), f32),
        jax.ShapeDtypeStruct((B, D), f32),
        jax.ShapeDtypeStruct((B,), f32),
        jax.ShapeDtypeStruct((B,), f32),
        jax.ShapeDtypeStruct((B,), f32),
    )
    scratch = (
        [pltpu.VMEM((CHUNK,), jnp.int32) for _ in range(3)]
        + [pltpu.VMEM((CHUNK, D), f32) for _ in range(3)]
        + [pltpu.VMEM((CHUNK,), f32) for _ in range(3)]
        + [pltpu.SemaphoreType.DMA, pltpu.SemaphoreType.DMA]
    )

    @functools.partial(pl.kernel, mesh=mesh, out_type=out_types,
                       scratch_types=scratch)
    def k(u3_h, ea_h, ec_h, lu_h, la_h, lc_h,
          iu_h, ia_h, ic_h,
          oeu, oea, oec, olu, ola, olc,
          viu, via, vic, veu, vea, vec, vlu, vla, vlc, sem, lsem):
        wid = lax.axis_index("s") * NC + lax.axis_index("c")
        base = wid * BPW
        for c in range(NCHUNK):
            off = base + c * CHUNK
            pltpu.sync_copy(iu_h.at[pl.ds(off, CHUNK)], viu)
            pltpu.sync_copy(ia_h.at[pl.ds(off, CHUNK)], via)
            pltpu.sync_copy(ic_h.at[pl.ds(off, CHUNK)], vic)
            hl = [pltpu.async_copy(lu_h.at[viu], vlu, lsem),
                  pltpu.async_copy(la_h.at[via], vla, lsem),
                  pltpu.async_copy(lc_h.at[vic], vlc, lsem)]

            @pl.loop(0, CHUNK // G)
            def _(g):
                vu = viu[pl.ds(g * G, G)]
                va = via[pl.ds(g * G, G)]
                vc = vic[pl.ds(g * G, G)]
                for j in range(G):
                    p = g * G + j
                    iu = vu[j]
                    hi = (iu >= HALF).astype(jnp.int32)
                    r2 = iu - hi * HALF
                    pltpu.async_copy(u3_h.at[hi].at[pl.ds(r2, 1)],
                                     veu.at[pl.ds(p, 1)], sem)
                    pltpu.async_copy(
                        ea_h.at[pl.ds(va[j], 1)], vea.at[pl.ds(p, 1)], sem)
                    pltpu.async_copy(
                        ec_h.at[pl.ds(vc[j], 1)], vec.at[pl.ds(p, 1)], sem)

            pltpu.make_async_copy(ea_h.at[pl.ds(0, CHUNK)], veu, sem).wait()
            pltpu.make_async_copy(ea_h.at[pl.ds(0, CHUNK)], vea, sem).wait()
            pltpu.make_async_copy(ea_h.at[pl.ds(0, CHUNK)], vec, sem).wait()
            for h in hl:
                h.wait()
            pltpu.sync_copy(veu, oeu.at[pl.ds(off, CHUNK)])
            pltpu.sync_copy(vea, oea.at[pl.ds(off, CHUNK)])
            pltpu.sync_copy(vec, oec.at[pl.ds(off, CHUNK)])
            pltpu.sync_copy(vlu, olu.at[pl.ds(off, CHUNK)])
            pltpu.sync_copy(vla, ola.at[pl.ds(off, CHUNK)])
            pltpu.sync_copy(vlc, olc.at[pl.ds(off, CHUNK)])

    return k(u3, emb_a, emb_c, lin_u1, lin_a1, lin_c1,
             idx_u, idx_a, idx_c)


def _tc_body(eu_r, ea_r, ec_r, lt_r,
             W1_r, b1_r, W2_r, b2_r, w3_r, off_r, o_r):
    eu = eu_r[...]
    ea = ea_r[...]
    ec = ec_r[...]
    s = eu + ea + ec
    sum_sq = jnp.sum(s * s, axis=1, keepdims=True)
    sq_sum = jnp.sum(eu * eu + ea * ea + ec * ec, axis=1, keepdims=True)
    interaction = 0.5 * (sum_sq - sq_sum)
    W1 = W1_r[...]
    f32 = jnp.float32
    h = (jnp.dot(eu, W1[0:D, :], preferred_element_type=f32)
         + jnp.dot(ea, W1[D:2 * D, :], preferred_element_type=f32)
         + jnp.dot(ec, W1[2 * D:3 * D, :], preferred_element_type=f32)
         + b1_r[...])
    h = jnp.maximum(h, 0.0)
    h = jnp.dot(h, W2_r[...], preferred_element_type=f32) + b2_r[...]
    h = jnp.maximum(h, 0.0)
    deep = jnp.sum(h * w3_r[...], axis=1, keepdims=True)
    o_r[...] = jax.nn.sigmoid(off_r[...] + lt_r[...] + interaction + deep)


def _tc_dense(eu, ea, ec, lt, W1, b1, W2, b2, w3, off):
    grid = (B // BLK,)
    bs_emb = pl.BlockSpec((BLK, D), lambda i: (i, 0))
    bs_col = pl.BlockSpec((BLK, 1), lambda i: (i, 0))
    rep = lambda shape: pl.BlockSpec(shape, lambda i: (0, 0))
    return pl.pallas_call(
        _tc_body,
        grid=grid,
        in_specs=[
            bs_emb, bs_emb, bs_emb, bs_col,
            rep((3 * D, 128)), rep((1, 128)),
            rep((128, D)), rep((1, D)),
            rep((1, D)), rep((1, 1)),
        ],
        out_specs=pl.BlockSpec((BLK, 1), lambda i: (i, 0)),
        out_shape=jax.ShapeDtypeStruct((B, 1), jnp.float32),
    )(eu, ea, ec, lt, W1, b1, W2, b2, w3, off)


BLK = 1024


def kernel(user_id, ad_id, context_id, lin_user, lin_ad, lin_ctx,
           emb_user, emb_ad, emb_ctx, bias, W1, b1, W2, b2, W3, b3):
    lu1 = lin_user.reshape(-1)
    la1 = lin_ad.reshape(-1)
    lc1 = lin_ctx.reshape(-1)
    uid = user_id.astype(jnp.int32)
    aid = ad_id.astype(jnp.int32)
    cid = context_id.astype(jnp.int32)

    u3 = emb_user.reshape(2, HALF, D)

    geu, gea, gec, glu, gla, glc = _sc_gather(
        u3, emb_ad, emb_ctx, lu1, la1, lc1, uid, aid, cid)

    lt = (glu + gla + glc).reshape(B, 1)
    off = (bias + b3).reshape(1, 1)
    out = _tc_dense(geu, gea, gec, lt,
                    W1, b1.reshape(1, 128), W2, b2.reshape(1, D),
                    W3.reshape(1, D), off)
    return out.reshape(B)
